# Initial kernel scaffold; baseline (speedup 1.0000x reference)
import jax
import jax.numpy as jnp
from jax.experimental import pallas as pl


def kernel(input_ids, token_type_ids, word_embeddings, position_embeddings, token_type_embeddings, ln_gamma, ln_beta):
    raise NotImplementedError("write your pallas kernel here")


# speedup vs baseline: 1.9964x; 1.9964x over previous
import functools

import jax
import jax.numpy as jnp
from jax import lax
from jax.experimental import pallas as pl
from jax.experimental.pallas import tpu as pltpu
from jax.experimental.pallas import tpu_sc as plsc

B = 256
S = 512
H = 768
N = B * S
LN_EPS = 1e-12

NC = 2
NS = 16
NW = NC * NS
PER_W = N // NW
CH = 128
NCHUNK = PER_W // CH

_sc_mesh = plsc.VectorSubcoreMesh(core_axis_name="c", subcore_axis_name="s")


@functools.partial(
    pl.kernel,
    mesh=_sc_mesh,
    out_type=jax.ShapeDtypeStruct((N, H), jnp.float32),
    scratch_types=[
        pltpu.VMEM((CH,), jnp.int32),
        pltpu.VMEM((CH, H), jnp.float32),
        pltpu.SemaphoreType.DMA,
    ],
)
def _sc_gather(idx_hbm, table_hbm, out_hbm, idx_v, rows_v, sem):
    wid = lax.axis_index("s") * NC + lax.axis_index("c")
    base = wid * PER_W

    def body(c, carry):
        off = base + c * CH
        pltpu.sync_copy(idx_hbm.at[pl.ds(off, CH)], idx_v)
        pltpu.async_copy(table_hbm.at[idx_v], rows_v, sem).wait()
        pltpu.sync_copy(rows_v, out_hbm.at[pl.ds(off, CH)])
        return carry

    lax.fori_loop(0, NCHUNK, body, 0)


def _ln_body(word_ref, pos_ref, tt_ref, tokemb_ref, gamma_ref, beta_ref, out_ref):
    word = word_ref[0]
    tok0 = tokemb_ref[0:1, :]
    tok1 = tokemb_ref[1:2, :]
    tt = tt_ref[0]
    emb = word + pos_ref[...] + tok0 + tt * (tok1 - tok0)
    mean = jnp.mean(emb, axis=-1, keepdims=True)
    d = emb - mean
    var = jnp.mean(d * d, axis=-1, keepdims=True)
    out_ref[0] = (d * lax.rsqrt(var + LN_EPS)) * gamma_ref[...] + beta_ref[...]


def _tc_layernorm(gathered, token_type_f32, pos_emb, tok_emb, gamma, beta):
    return pl.pallas_call(
        _ln_body,
        grid=(B,),
        in_specs=[
            pl.BlockSpec((1, S, H), lambda b: (b, 0, 0)),
            pl.BlockSpec((S, H), lambda b: (0, 0)),
            pl.BlockSpec((1, S, 1), lambda b: (b, 0, 0)),
            pl.BlockSpec((2, H), lambda b: (0, 0)),
            pl.BlockSpec((1, H), lambda b: (0, 0)),
            pl.BlockSpec((1, H), lambda b: (0, 0)),
        ],
        out_specs=pl.BlockSpec((1, S, H), lambda b: (b, 0, 0)),
        out_shape=jax.ShapeDtypeStruct((B, S, H), jnp.float32),
    )(gathered, pos_emb, token_type_f32, tok_emb, gamma, beta)


def kernel(input_ids, token_type_ids, word_embeddings, position_embeddings,
           token_type_embeddings, ln_gamma, ln_beta):
    ids = input_ids.reshape(N).astype(jnp.int32)
    gathered = _sc_gather(ids, word_embeddings).reshape(B, S, H)
    tt = token_type_ids.astype(jnp.float32).reshape(B, S, 1)
    return _tc_layernorm(
        gathered, tt, position_embeddings, token_type_embeddings,
        ln_gamma.reshape(1, H), ln_beta.reshape(1, H))


# speedup vs baseline: 2.0588x; 1.0313x over previous
import functools

import jax
import jax.numpy as jnp
from jax import lax
from jax.experimental import pallas as pl
from jax.experimental.pallas import tpu as pltpu
from jax.experimental.pallas import tpu_sc as plsc

B = 256
S = 512
H = 768
N = B * S
LN_EPS = 1e-12

NC = 2
NS = 16
NW = NC * NS
PER_W = N // NW
CH = 64
NCHUNK = PER_W // CH
NPAIR = NCHUNK // 2

_sc_mesh = plsc.VectorSubcoreMesh(core_axis_name="c", subcore_axis_name="s")


@functools.partial(
    pl.kernel,
    mesh=_sc_mesh,
    out_type=jax.ShapeDtypeStruct((N, H), jnp.float32),
    scratch_types=[
        pltpu.VMEM((NCHUNK, CH), jnp.int32),
        pltpu.VMEM((CH, H), jnp.float32),
        pltpu.VMEM((CH, H), jnp.float32),
        pltpu.SemaphoreType.DMA,
        pltpu.SemaphoreType.DMA,
        pltpu.SemaphoreType.DMA,
        pltpu.SemaphoreType.DMA,
    ],
)
def _sc_gather(idx_hbm, table_hbm, out_hbm, idx_all, buf0, buf1,
               gsem0, gsem1, ssem0, ssem1):
    wid = lax.axis_index("s") * NC + lax.axis_index("c")
    base = wid * PER_W
    pltpu.sync_copy(idx_hbm.at[wid], idx_all)

    def start_gather(c, buf, gsem):
        pltpu.async_copy(table_hbm.at[idx_all.at[c]], buf, gsem)

    def wait_gather(buf, gsem):
        pltpu.make_async_copy(table_hbm.at[idx_all.at[0]], buf, gsem).wait()

    def start_store(c, buf, ssem):
        pltpu.async_copy(buf, out_hbm.at[pl.ds(base + c * CH, CH)], ssem)

    def wait_store(buf, ssem):
        pltpu.make_async_copy(buf, out_hbm.at[pl.ds(base, CH)], ssem).wait()

    start_gather(0, buf0, gsem0)

    def body(p, carry):
        c0 = 2 * p
        c1 = c0 + 1
        wait_gather(buf0, gsem0)
        start_store(c0, buf0, ssem0)

        @pl.when(p > 0)
        def _():
            wait_store(buf1, ssem1)
        start_gather(c1, buf1, gsem1)
        wait_gather(buf1, gsem1)
        start_store(c1, buf1, ssem1)

        @pl.when(p + 1 < NPAIR)
        def _():
            wait_store(buf0, ssem0)
            start_gather(c0 + 2, buf0, gsem0)
        return carry

    lax.fori_loop(0, NPAIR, body, 0)
    wait_store(buf0, ssem0)
    wait_store(buf1, ssem1)


def _ln_body(word_ref, pos_ref, tt_ref, tokemb_ref, gamma_ref, beta_ref, out_ref):
    word = word_ref[0]
    tok0 = tokemb_ref[0:1, :]
    tok1 = tokemb_ref[1:2, :]
    tt = tt_ref[0]
    emb = word + pos_ref[...] + tok0 + tt * (tok1 - tok0)
    mean = jnp.mean(emb, axis=-1, keepdims=True)
    d = emb - mean
    var = jnp.mean(d * d, axis=-1, keepdims=True)
    out_ref[0] = (d * lax.rsqrt(var + LN_EPS)) * gamma_ref[...] + beta_ref[...]


def _tc_layernorm(gathered, token_type_f32, pos_emb, tok_emb, gamma, beta):
    return pl.pallas_call(
        _ln_body,
        grid=(B,),
        in_specs=[
            pl.BlockSpec((1, S, H), lambda b: (b, 0, 0)),
            pl.BlockSpec((S, H), lambda b: (0, 0)),
            pl.BlockSpec((1, S, 1), lambda b: (b, 0, 0)),
            pl.BlockSpec((2, H), lambda b: (0, 0)),
            pl.BlockSpec((1, H), lambda b: (0, 0)),
            pl.BlockSpec((1, H), lambda b: (0, 0)),
        ],
        out_specs=pl.BlockSpec((1, S, H), lambda b: (b, 0, 0)),
        out_shape=jax.ShapeDtypeStruct((B, S, H), jnp.float32),
    )(gathered, pos_emb, token_type_f32, tok_emb, gamma, beta)


def kernel(input_ids, token_type_ids, word_embeddings, position_embeddings,
           token_type_embeddings, ln_gamma, ln_beta):
    ids = input_ids.reshape(NW, NCHUNK, CH).astype(jnp.int32)
    gathered = _sc_gather(ids, word_embeddings).reshape(B, S, H)
    tt = token_type_ids.astype(jnp.float32).reshape(B, S, 1)
    return _tc_layernorm(
        gathered, tt, position_embeddings, token_type_embeddings,
        ln_gamma.reshape(1, H), ln_beta.reshape(1, H))


# speedup vs baseline: 2.2214x; 1.0790x over previous
import functools

import jax
import jax.numpy as jnp
from jax import lax
from jax.experimental import pallas as pl
from jax.experimental.pallas import tpu as pltpu
from jax.experimental.pallas import tpu_sc as plsc

B = 256
S = 512
H = 768
N = B * S
LN_EPS = 1e-12

G = 4
BG = B // G
NG = BG * S

NC = 2
NS = 16
NW = NC * NS
PER_W = NG // NW
CH = 64
NCHUNK = PER_W // CH
NPAIR = NCHUNK // 2

_sc_mesh = plsc.VectorSubcoreMesh(core_axis_name="c", subcore_axis_name="s")


@functools.partial(
    pl.kernel,
    mesh=_sc_mesh,
    out_type=jax.ShapeDtypeStruct((NG, H), jnp.float32),
    scratch_types=[
        pltpu.VMEM((NCHUNK, CH), jnp.int32),
        pltpu.VMEM((CH, H), jnp.float32),
        pltpu.VMEM((CH, H), jnp.float32),
        pltpu.SemaphoreType.DMA,
        pltpu.SemaphoreType.DMA,
        pltpu.SemaphoreType.DMA,
        pltpu.SemaphoreType.DMA,
    ],
)
def _sc_gather(idx_hbm, table_hbm, out_hbm, idx_all, buf0, buf1,
               gsem0, gsem1, ssem0, ssem1):
    wid = lax.axis_index("s") * NC + lax.axis_index("c")
    base = wid * PER_W
    pltpu.sync_copy(idx_hbm.at[wid], idx_all)

    def start_gather(c, buf, gsem):
        pltpu.async_copy(table_hbm.at[idx_all.at[c]], buf, gsem)

    def wait_gather(buf, gsem):
        pltpu.make_async_copy(table_hbm.at[idx_all.at[0]], buf, gsem).wait()

    def start_store(c, buf, ssem):
        pltpu.async_copy(buf, out_hbm.at[pl.ds(base + c * CH, CH)], ssem)

    def wait_store(buf, ssem):
        pltpu.make_async_copy(buf, out_hbm.at[pl.ds(base, CH)], ssem).wait()

    start_gather(0, buf0, gsem0)

    def body(p, carry):
        c0 = 2 * p
        c1 = c0 + 1
        wait_gather(buf0, gsem0)
        start_store(c0, buf0, ssem0)

        @pl.when(p > 0)
        def _():
            wait_store(buf1, ssem1)
        start_gather(c1, buf1, gsem1)
        wait_gather(buf1, gsem1)
        start_store(c1, buf1, ssem1)

        @pl.when(p + 1 < NPAIR)
        def _():
            wait_store(buf0, ssem0)
            start_gather(c0 + 2, buf0, gsem0)
        return carry

    lax.fori_loop(0, NPAIR, body, 0)
    wait_store(buf0, ssem0)
    wait_store(buf1, ssem1)


def _ln_compute(word_ref, pos_ref, tt_ref, tokemb_ref, gamma_ref, beta_ref,
                out_ref):
    word = word_ref[0]
    tok0 = tokemb_ref[0:1, :]
    tok1 = tokemb_ref[1:2, :]
    tt = tt_ref[0]
    emb = word + pos_ref[...] + tok0 + tt * (tok1 - tok0)
    mean = jnp.mean(emb, axis=-1, keepdims=True)
    d = emb - mean
    var = jnp.mean(d * d, axis=-1, keepdims=True)
    out_ref[0] = (d * lax.rsqrt(var + LN_EPS)) * gamma_ref[...] + beta_ref[...]


def _ln_body_first(word_ref, pos_ref, tt_ref, tokemb_ref, gamma_ref, beta_ref,
                   out_ref):
    _ln_compute(word_ref, pos_ref, tt_ref, tokemb_ref, gamma_ref, beta_ref,
                out_ref)


def _ln_body_chained(prev_ref, word_ref, pos_ref, tt_ref, tokemb_ref,
                     gamma_ref, beta_ref, out_ref):
    del prev_ref
    _ln_compute(word_ref, pos_ref, tt_ref, tokemb_ref, gamma_ref, beta_ref,
                out_ref)


def _tc_ln_chunk(g, prev_out, gathered, tt, pos_emb, tok_emb, gamma, beta):
    common_in_specs = [
        pl.BlockSpec((1, S, H), lambda b: (b, 0, 0)),
        pl.BlockSpec((S, H), lambda b: (0, 0)),
        pl.BlockSpec((1, S, 1), lambda b, g=g: (g * BG + b, 0, 0)),
        pl.BlockSpec((2, H), lambda b: (0, 0)),
        pl.BlockSpec((1, H), lambda b: (0, 0)),
        pl.BlockSpec((1, H), lambda b: (0, 0)),
    ]
    out_spec = pl.BlockSpec((1, S, H), lambda b, g=g: (g * BG + b, 0, 0))
    out_shape = jax.ShapeDtypeStruct((B, S, H), jnp.float32)
    if prev_out is None:
        return pl.pallas_call(
            _ln_body_first,
            grid=(BG,),
            in_specs=common_in_specs,
            out_specs=out_spec,
            out_shape=out_shape,
        )(gathered, pos_emb, tt, tok_emb, gamma, beta)
    return pl.pallas_call(
        _ln_body_chained,
        grid=(BG,),
        in_specs=[pl.BlockSpec(memory_space=pl.ANY)] + common_in_specs,
        out_specs=out_spec,
        out_shape=out_shape,
        input_output_aliases={0: 0},
    )(prev_out, gathered, pos_emb, tt, tok_emb, gamma, beta)


def kernel(input_ids, token_type_ids, word_embeddings, position_embeddings,
           token_type_embeddings, ln_gamma, ln_beta):
    ids = input_ids.reshape(G, NW, NCHUNK, CH).astype(jnp.int32)
    tt = token_type_ids.astype(jnp.float32).reshape(B, S, 1)
    gamma = ln_gamma.reshape(1, H)
    beta = ln_beta.reshape(1, H)
    out = None
    for g in range(G):
        gathered = _sc_gather(ids[g], word_embeddings).reshape(BG, S, H)
        out = _tc_ln_chunk(g, out, gathered, tt, position_embeddings,
                           token_type_embeddings, gamma, beta)
    return out


# speedup vs baseline: 2.5405x; 1.1437x over previous
import functools

import jax
import jax.numpy as jnp
from jax import lax
from jax.experimental import pallas as pl
from jax.experimental.pallas import tpu as pltpu
from jax.experimental.pallas import tpu_sc as plsc

B = 256
S = 512
H = 768
HP = H // 2
N = B * S
LN_EPS = 1e-12

G = 4
BG = B // G
NG = BG * S

NC = 2
NS = 16
NW = NC * NS
PER_W = NG // NW
CH = 128
NCHUNK = PER_W // CH
NPAIR = NCHUNK // 2

_sc_mesh = plsc.VectorSubcoreMesh(core_axis_name="c", subcore_axis_name="s")


@functools.partial(
    pl.kernel,
    mesh=_sc_mesh,
    out_type=jax.ShapeDtypeStruct((NG, HP), jnp.int32),
    scratch_types=[
        pltpu.VMEM((NCHUNK, CH), jnp.int32),
        pltpu.VMEM((CH, HP), jnp.int32),
        pltpu.VMEM((CH, HP), jnp.int32),
        pltpu.SemaphoreType.DMA,
        pltpu.SemaphoreType.DMA,
        pltpu.SemaphoreType.DMA,
        pltpu.SemaphoreType.DMA,
    ],
)
def _sc_gather(idx_hbm, table_hbm, out_hbm, idx_all, buf0, buf1,
               gsem0, gsem1, ssem0, ssem1):
    wid = lax.axis_index("s") * NC + lax.axis_index("c")
    base = wid * PER_W
    pltpu.sync_copy(idx_hbm.at[wid], idx_all)

    def start_gather(c, buf, gsem):
        pltpu.async_copy(table_hbm.at[idx_all.at[c]], buf, gsem)

    def wait_gather(buf, gsem):
        pltpu.make_async_copy(table_hbm.at[idx_all.at[0]], buf, gsem).wait()

    def start_store(c, buf, ssem):
        pltpu.async_copy(buf, out_hbm.at[pl.ds(base + c * CH, CH)], ssem)

    def wait_store(buf, ssem):
        pltpu.make_async_copy(buf, out_hbm.at[pl.ds(base, CH)], ssem).wait()

    start_gather(0, buf0, gsem0)

    def body(p, carry):
        c0 = 2 * p
        c1 = c0 + 1
        wait_gather(buf0, gsem0)
        start_store(c0, buf0, ssem0)

        @pl.when(p > 0)
        def _():
            wait_store(buf1, ssem1)
        start_gather(c1, buf1, gsem1)
        wait_gather(buf1, gsem1)
        start_store(c1, buf1, ssem1)

        @pl.when(p + 1 < NPAIR)
        def _():
            wait_store(buf0, ssem0)
            start_gather(c0 + 2, buf0, gsem0)
        return carry

    lax.fori_loop(0, NPAIR, body, 0)
    wait_store(buf0, ssem0)
    wait_store(buf1, ssem1)


def _ln_compute(word_ref, pos_ref, tt_ref, tokemb_ref, gamma_ref, beta_ref,
                out_ref):
    w = word_ref[0]
    word_lo = lax.bitcast_convert_type(w << 16, jnp.float32)
    word_hi = lax.bitcast_convert_type(w & jnp.int32(-65536), jnp.float32)
    tt = tt_ref[0]
    tok0 = tokemb_ref[0:1, :]
    tok1 = tokemb_ref[1:2, :]
    emb_lo = word_lo + pos_ref[:, 0:HP] + tok0[:, 0:HP] + tt * (tok1[:, 0:HP] - tok0[:, 0:HP])
    emb_hi = word_hi + pos_ref[:, HP:H] + tok0[:, HP:H] + tt * (tok1[:, HP:H] - tok0[:, HP:H])
    mean = (jnp.sum(emb_lo, axis=-1, keepdims=True)
            + jnp.sum(emb_hi, axis=-1, keepdims=True)) * (1.0 / H)
    d_lo = emb_lo - mean
    d_hi = emb_hi - mean
    var = (jnp.sum(d_lo * d_lo, axis=-1, keepdims=True)
           + jnp.sum(d_hi * d_hi, axis=-1, keepdims=True)) * (1.0 / H)
    rstd = lax.rsqrt(var + LN_EPS)
    out_ref[0, :, 0:HP] = (d_lo * rstd) * gamma_ref[:, 0:HP] + beta_ref[:, 0:HP]
    out_ref[0, :, HP:H] = (d_hi * rstd) * gamma_ref[:, HP:H] + beta_ref[:, HP:H]


def _ln_body_first(word_ref, pos_ref, tt_ref, tokemb_ref, gamma_ref, beta_ref,
                   out_ref):
    _ln_compute(word_ref, pos_ref, tt_ref, tokemb_ref, gamma_ref, beta_ref,
                out_ref)


def _ln_body_chained(prev_ref, word_ref, pos_ref, tt_ref, tokemb_ref,
                     gamma_ref, beta_ref, out_ref):
    del prev_ref
    _ln_compute(word_ref, pos_ref, tt_ref, tokemb_ref, gamma_ref, beta_ref,
                out_ref)


def _tc_ln_chunk(g, prev_out, gathered, tt, pos_emb, tok_emb, gamma, beta):
    common_in_specs = [
        pl.BlockSpec((1, S, HP), lambda b: (b, 0, 0)),
        pl.BlockSpec((S, H), lambda b: (0, 0)),
        pl.BlockSpec((1, S, 1), lambda b, g=g: (g * BG + b, 0, 0)),
        pl.BlockSpec((2, H), lambda b: (0, 0)),
        pl.BlockSpec((1, H), lambda b: (0, 0)),
        pl.BlockSpec((1, H), lambda b: (0, 0)),
    ]
    out_spec = pl.BlockSpec((1, S, H), lambda b, g=g: (g * BG + b, 0, 0))
    out_shape = jax.ShapeDtypeStruct((B, S, H), jnp.float32)
    if prev_out is None:
        return pl.pallas_call(
            _ln_body_first,
            grid=(BG,),
            in_specs=common_in_specs,
            out_specs=out_spec,
            out_shape=out_shape,
        )(gathered, pos_emb, tt, tok_emb, gamma, beta)
    return pl.pallas_call(
        _ln_body_chained,
        grid=(BG,),
        in_specs=[pl.BlockSpec(memory_space=pl.ANY)] + common_in_specs,
        out_specs=out_spec,
        out_shape=out_shape,
        input_output_aliases={0: 0},
    )(prev_out, gathered, pos_emb, tt, tok_emb, gamma, beta)


def _pack_table(word_embeddings):
    wb = word_embeddings.astype(jnp.bfloat16)
    lo = lax.bitcast_convert_type(wb[:, :HP], jnp.uint16).astype(jnp.uint32)
    hi = lax.bitcast_convert_type(wb[:, HP:], jnp.uint16).astype(jnp.uint32)
    return lax.bitcast_convert_type(lo | (hi << 16), jnp.int32)


def kernel(input_ids, token_type_ids, word_embeddings, position_embeddings,
           token_type_embeddings, ln_gamma, ln_beta):
    ids = input_ids.reshape(G, NW, NCHUNK, CH).astype(jnp.int32)
    tt = token_type_ids.astype(jnp.float32).reshape(B, S, 1)
    table_packed = _pack_table(word_embeddings)
    gamma = ln_gamma.reshape(1, H)
    beta = ln_beta.reshape(1, H)
    out = None
    for g in range(G):
        gathered = _sc_gather(ids[g], table_packed).reshape(BG, S, HP)
        out = _tc_ln_chunk(g, out, gathered, tt, position_embeddings,
                           token_type_embeddings, gamma, beta)
    return out
